# Initial kernel scaffold; baseline (speedup 1.0000x reference)
import jax
import jax.numpy as jnp
from jax.experimental import pallas as pl


def kernel(itemFeatures, table, W1, b1, W2, b2):
    raise NotImplementedError("write your pallas kernel here")


# speedup vs baseline: 16.7225x; 16.7225x over previous
import functools

import jax
import jax.numpy as jnp
from jax import lax
from jax.experimental import pallas as pl
from jax.experimental.pallas import tpu as pltpu
from jax.experimental.pallas import tpu_sc as plsc

_VOCAB = 1000000
_D = 32
_F = 26
_B = 16384
_H = 256
_ALL = _F * _D

_NC = 2
_NS = 16
_NW = _NC * _NS
_ROWS = _B * _F
_RPW = _ROWS // _NW
_CHUNK = 128
_NCH = _RPW // _CHUNK
_NBUF = 2


def _gather_body(idx_hbm, table_hbm, out_hbm, idx_v, rows_v, gsem, osem):
    wid = lax.axis_index("s") * _NC + lax.axis_index("c")
    base = wid * _RPW
    pltpu.sync_copy(idx_hbm.at[wid], idx_v)

    pltpu.async_copy(table_hbm.at[idx_v.at[0]], rows_v.at[0], gsem)

    def body(j, _):
        slot = lax.rem(j, _NBUF)
        nxt = lax.rem(j + 1, _NBUF)

        @pl.when(j + 1 < _NCH)
        def _start_next():
            pltpu.async_copy(table_hbm.at[idx_v.at[j + 1]], rows_v.at[nxt],
                             gsem)

        pltpu.make_async_copy(table_hbm.at[idx_v.at[j]], rows_v.at[slot],
                              gsem).wait()
        out_slice = out_hbm.at[pl.ds(base + j * _CHUNK, _CHUNK)]
        copy = pltpu.make_async_copy(rows_v.at[slot], out_slice, osem)
        copy.start()

        @pl.when(j >= 1)
        def _drain_prev():
            prev = lax.rem(j - 1, _NBUF)
            prev_slice = out_hbm.at[pl.ds(base + (j - 1) * _CHUNK, _CHUNK)]
            pltpu.make_async_copy(rows_v.at[prev], prev_slice, osem).wait()

        return 0

    lax.fori_loop(0, _NCH, body, 0)
    last_slice = out_hbm.at[pl.ds(base + (_NCH - 1) * _CHUNK, _CHUNK)]
    pltpu.make_async_copy(rows_v.at[(_NCH - 1) % _NBUF], last_slice,
                          osem).wait()


@functools.partial(
    pl.kernel,
    mesh=plsc.VectorSubcoreMesh(core_axis_name="c", subcore_axis_name="s"),
    compiler_params=pltpu.CompilerParams(use_tc_tiling_on_sc=False),
    out_type=jax.ShapeDtypeStruct((_ROWS, _D), jnp.float32),
    scratch_types=[
        pltpu.VMEM((_NCH, _CHUNK), jnp.int32),
        pltpu.VMEM((_NBUF, _CHUNK, _D), jnp.float32),
        pltpu.SemaphoreType.DMA,
        pltpu.SemaphoreType.DMA,
    ],
)
def _sc_gather(idx_hbm, table_hbm, out_hbm, idx_v, rows_v, gsem, osem):
    _gather_body(idx_hbm, table_hbm, out_hbm, idx_v, rows_v, gsem, osem)


_BB = 1024


def _mlp_body(emb_ref, w1_ref, b1_ref, w2_ref, b2_ref, out_ref):
    h = jnp.dot(emb_ref[...], w1_ref[...],
                preferred_element_type=jnp.float32)
    h = jnp.maximum(h + b1_ref[...], 0.0)
    out_ref[...] = jnp.dot(h, w2_ref[...],
                           preferred_element_type=jnp.float32) + b2_ref[...]


def _mlp(emb, W1, b1, W2, b2):
    return pl.pallas_call(
        _mlp_body,
        grid=(_B // _BB,),
        in_specs=[
            pl.BlockSpec((_BB, _ALL), lambda i: (i, 0)),
            pl.BlockSpec((_ALL, _H), lambda i: (0, 0)),
            pl.BlockSpec((1, _H), lambda i: (0, 0)),
            pl.BlockSpec((_H, _D), lambda i: (0, 0)),
            pl.BlockSpec((1, _D), lambda i: (0, 0)),
        ],
        out_specs=pl.BlockSpec((_BB, _D), lambda i: (i, 0)),
        out_shape=jax.ShapeDtypeStruct((_B, _D), jnp.float32),
    )(emb, W1, b1, W2, b2)


def kernel(itemFeatures, table, W1, b1, W2, b2):
    idx = itemFeatures.reshape(_NW, _NCH, _CHUNK)
    emb_flat = _sc_gather(idx, table)
    emb = emb_flat.reshape(_B, _ALL)
    return _mlp(emb, W1, b1.reshape(1, _H), W2, b2.reshape(1, _D))
